# baseline (speedup 1.0000x reference)
import functools

import jax
import jax.numpy as jnp
from jax import lax
from jax.experimental import pallas as pl
from jax.experimental.pallas import tpu as pltpu
from jax.experimental.pallas import tpu_sc as plsc

N = 10000
D = 128
E = 320000
NC = 2
NS = 16
NW = NC * NS
EPW = E // NW
CH = 80
NCHK = EPW // CH
NPAD = 10240
RPS = NPAD // NS
ZR = 128
DEGN = 10240
RPSD = DEGN // NS


def _mesh():
    return plsc.VectorSubcoreMesh(
        core_axis_name="c", subcore_axis_name="s",
        num_cores=NC, num_subcores=NS)


def _make_deg():
    @functools.partial(
        pl.kernel,
        out_type=jax.ShapeDtypeStruct((NC, DEGN, D), jnp.float32),
        mesh=_mesh(),
        scratch_types=[
            pltpu.VMEM_SHARED((DEGN, D), jnp.float32),
            pltpu.VMEM((NCHK, CH), jnp.int32),
            pltpu.VMEM((CH, D), jnp.float32),
            pltpu.SemaphoreType.DMA,
        ],
    )
    def deg_k(dst_hbm, ones_hbm, zeros_hbm, out_hbm, acc, idx_v, ones_v,
              dsem):
        c = lax.axis_index("c")
        s = lax.axis_index("s")
        wid = c * NS + s
        pltpu.sync_copy(dst_hbm.at[wid], idx_v)
        pltpu.sync_copy(ones_hbm, ones_v)
        pltpu.sync_copy(zeros_hbm, acc.at[pl.ds(s * RPSD, RPSD)])
        plsc.subcore_barrier()

        def body(j, carry):
            pltpu.async_copy(ones_v, acc.at[idx_v.at[j]], dsem, add=True)
            return carry

        lax.fori_loop(0, NCHK, body, 0)

        def drain(j, carry):
            pltpu.make_async_copy(ones_v, acc.at[pl.ds(0, CH)], dsem).wait()
            return carry

        lax.fori_loop(0, NCHK, drain, 0)
        plsc.subcore_barrier()
        pltpu.sync_copy(acc.at[pl.ds(s * RPSD, RPSD)],
                        out_hbm.at[c, pl.ds(s * RPSD, RPSD)])

    return deg_k


NBUF = 3
RING = 6


def _make_spmm(F):
    @functools.partial(
        pl.kernel,
        out_type=jax.ShapeDtypeStruct((NC, F, NPAD, D), jnp.float32),
        mesh=_mesh(),
        scratch_types=[
            pltpu.VMEM_SHARED((NPAD, D), jnp.float32),
            pltpu.VMEM((RING, CH), jnp.int32),
            pltpu.VMEM((NCHK, CH), jnp.int32),
            pltpu.VMEM((NBUF, CH, D), jnp.float32),
            pltpu.SemaphoreType.DMA((NBUF,)),
            pltpu.SemaphoreType.DMA((NBUF,)),
            pltpu.SemaphoreType.DMA((RING,)),
        ],
    )
    def spmm_k(*args):
        hs = args[:F]
        src_hbm, dst_hbm, zeros_hbm, out_hbm = args[F:F + 4]
        acc, sring, dst_v, gbuf, gsem, ssem, isem = args[F + 4:]
        c = lax.axis_index("c")
        s = lax.axis_index("s")
        wid = c * NS + s
        pltpu.sync_copy(dst_hbm.at[wid], dst_v)

        def stage(i, slot):
            pltpu.async_copy(src_hbm.at[wid, pl.ds(i, 1)],
                             sring.at[pl.ds(slot, 1)], isem.at[slot])

        def wait_idx(slot):
            pltpu.make_async_copy(src_hbm.at[wid, pl.ds(0, 1)],
                                  sring.at[pl.ds(slot, 1)],
                                  isem.at[slot]).wait()

        def gather(h, slot, b):
            pltpu.async_copy(h.at[sring.at[slot]], gbuf.at[b], gsem.at[b])

        def wait_gather(h, b):
            pltpu.make_async_copy(h.at[pl.ds(0, CH)], gbuf.at[b],
                                  gsem.at[b]).wait()

        def scatter(i, b):
            pltpu.async_copy(gbuf.at[b], acc.at[dst_v.at[i]], ssem.at[b],
                             add=True)

        def wait_scatter(b):
            pltpu.make_async_copy(gbuf.at[b], acc.at[pl.ds(0, CH)],
                                  ssem.at[b]).wait()

        def step(h, i, k, i_static=True):
            b = k % NBUF
            b2 = (k + 2) % NBUF
            r = k % RING
            r2 = (k + 2) % RING
            wait_gather(h, b)
            scatter(i, b)
            if not (i_static and i < 1):
                wait_scatter(b2)
            if not (i_static and i + 2 >= NCHK):
                wait_idx(r2)
                gather(h, r2, b2)
            if not (i_static and i + 6 >= NCHK):
                stage(i + 6, r)

        for f in range(F):
            h = hs[f]
            pltpu.sync_copy(zeros_hbm, acc.at[pl.ds(s * RPS, RPS)])
            plsc.subcore_barrier()
            for r in range(RING):
                stage(r, r)
            wait_idx(0)
            gather(h, 0, 0)
            wait_idx(1)
            gather(h, 1, 1)
            for i in range(RING):
                step(h, i, i)
            def block(i6, carry):
                for k in range(RING):
                    step(h, i6 * RING + k, k, i_static=False)
                return carry
            n_main = (NCHK - 12) // RING
            lax.fori_loop(1, 1 + n_main, block, 0)
            for i in range((1 + n_main) * RING, NCHK):
                step(h, i, i % RING)
            wait_scatter((NCHK - 1) % NBUF)
            plsc.subcore_barrier()
            pltpu.sync_copy(acc.at[pl.ds(s * RPS, RPS)],
                            out_hbm.at[c, f, pl.ds(s * RPS, RPS)])
            plsc.subcore_barrier()

    return spmm_k


BN = 400


def _dinv_block(deg_ref):
    deg = deg_ref[0, :, 0] + deg_ref[1, :, 0]
    return lax.rsqrt(jnp.maximum(deg, 1.0))[:, None]


def _tc0_body(x_ref, w_ref, b_ref, h_ref):
    h_ref[...] = jnp.dot(x_ref[...], w_ref[...],
                         preferred_element_type=jnp.float32) + b_ref[...]


def _tc0(x, Wcat, bcat):
    nb = N // BN
    return pl.pallas_call(
        _tc0_body,
        grid=(nb,),
        in_specs=[
            pl.BlockSpec((BN, D), lambda i: (i, 0)),
            pl.BlockSpec((D, 5 * D), lambda i: (0, 0)),
            pl.BlockSpec((1, 5 * D), lambda i: (0, 0)),
        ],
        out_specs=pl.BlockSpec((BN, 5 * D), lambda i: (i, 0)),
        out_shape=jax.ShapeDtypeStruct((N, 5 * D), jnp.float32),
    )(x, Wcat, bcat)


def _tc1_body(h_ref, deg_ref, g0, g1, g2, g3):
    dinv = _dinv_block(deg_ref)
    for f, ref in enumerate((g0, g1, g2, g3)):
        ref[...] = h_ref[:, f * D:(f + 1) * D] * dinv


def _tc1(H, degp):
    nb = N // BN
    return pl.pallas_call(
        _tc1_body,
        grid=(nb,),
        in_specs=[
            pl.BlockSpec((BN, 5 * D), lambda i: (i, 0)),
            pl.BlockSpec((NC, BN, D), lambda i: (0, i, 0)),
        ],
        out_specs=[pl.BlockSpec((BN, D), lambda i: (i, 0))] * 4,
        out_shape=[jax.ShapeDtypeStruct((N, D), jnp.float32)] * 4,
    )(H, degp)


def _tc2_body(h_ref, u_ref, deg_ref, hp_pre, lp_pre, p_ref, t_ref, g2a, g2b):
    dinv = _dinv_block(deg_ref)
    u = u_ref[0] + u_ref[1]
    hp_pre[...] = h_ref[:, 0:D] - dinv * u[0]
    lp_pre[...] = dinv * u[1]
    pv = h_ref[:, 2 * D:3 * D] - dinv * u[2]
    tv = dinv * u[3]
    p_ref[...] = pv
    t_ref[...] = tv
    g2a[...] = dinv * pv
    g2b[...] = dinv * tv


def _tc2(H, U1, degp):
    nb = N // BN
    return pl.pallas_call(
        _tc2_body,
        grid=(nb,),
        in_specs=[
            pl.BlockSpec((BN, 5 * D), lambda i: (i, 0)),
            pl.BlockSpec((NC, 4, BN, D), lambda i: (0, 0, i, 0)),
            pl.BlockSpec((NC, BN, D), lambda i: (0, i, 0)),
        ],
        out_specs=[pl.BlockSpec((BN, D), lambda i: (i, 0))] * 6,
        out_shape=[jax.ShapeDtypeStruct((N, D), jnp.float32)] * 6,
    )(H, U1, degp)


def _tc3_body(hp_ref, lp_ref, p_ref, t_ref, h_ref, u_ref, deg_ref,
              lw_ref, lb_ref, out_ref):
    dinv = _dinv_block(deg_ref)
    u = u_ref[0] + u_ref[1]
    h_hp = jnp.maximum(hp_ref[...], 0.0)
    h_lp = jnp.maximum(lp_ref[...], 0.0)
    h_hp2 = jnp.maximum(p_ref[...] - dinv * u[0], 0.0)
    h_lp2 = jnp.maximum(dinv * u[1], 0.0)
    h_i = jnp.maximum(h_ref[:, 4 * D:5 * D], 0.0)
    acc = jnp.zeros((BN, D), jnp.float32)
    for k, hk in enumerate((h_hp, h_lp, h_hp2, h_lp2, h_i)):
        score = jnp.dot(hk, lw_ref[:, k:k + 1],
                        preferred_element_type=jnp.float32) + lb_ref[0, k]
        acc = acc + jax.nn.sigmoid(score) * hk
    out_ref[...] = acc


def _tc3(hp_pre, lp_pre, p, t, H, U2, degp, LW, LB):
    nb = N // BN
    return pl.pallas_call(
        _tc3_body,
        grid=(nb,),
        in_specs=[pl.BlockSpec((BN, D), lambda i: (i, 0))] * 4
        + [
            pl.BlockSpec((BN, 5 * D), lambda i: (i, 0)),
            pl.BlockSpec((NC, 2, BN, D), lambda i: (0, 0, i, 0)),
            pl.BlockSpec((NC, BN, D), lambda i: (0, i, 0)),
            pl.BlockSpec((D, 8), lambda i: (0, 0)),
            pl.BlockSpec((1, 8), lambda i: (0, 0)),
        ],
        out_specs=pl.BlockSpec((BN, D), lambda i: (i, 0)),
        out_shape=jax.ShapeDtypeStruct((N, D), jnp.float32),
    )(hp_pre, lp_pre, p, t, H, U2, degp, LW, LB)


def kernel(x, edge_index,
           W_hp, b_hp, lw_hp, lb_hp,
           W_lp, b_lp, lw_lp, lb_lp,
           W_i, b_i, lw_i, lb_i,
           W_hp2, b_hp2, lw_hp2, lb_hp2,
           W_lp2, b_lp2, lw_lp2, lb_lp2):
    src = edge_index[0].reshape(NW, NCHK, CH)
    dst = edge_index[1].reshape(NW, NCHK, CH)
    Wcat = jnp.concatenate([W_hp, W_lp, W_hp2, W_lp2, W_i], axis=1)
    bcat = jnp.concatenate([b_hp, b_lp, b_hp2, b_lp2, b_i]).reshape(1, 5 * D)
    LW = jnp.concatenate(
        [lw_hp, lw_lp, lw_hp2, lw_lp2, lw_i, jnp.zeros((D, 3), jnp.float32)],
        axis=1)
    LB = jnp.concatenate(
        [lb_hp, lb_lp, lb_hp2, lb_lp2, lb_i,
         jnp.zeros((3,), jnp.float32)]).reshape(1, 8)
    ones_c = jnp.ones((CH, D), jnp.float32)
    zeros_d = jnp.zeros((RPSD, D), jnp.float32)
    zeros_f = jnp.zeros((RPS, D), jnp.float32)

    degp = _make_deg()(dst, ones_c, zeros_d)
    H = _tc0(x, Wcat, bcat)
    g0, g1, g2, g3 = _tc1(H, degp)
    U1 = _make_spmm(4)(g0, g1, g2, g3, src, dst, zeros_f)
    hp_pre, lp_pre, p, t, g2a, g2b = _tc2(H, U1, degp)
    U2 = _make_spmm(2)(g2a, g2b, src, dst, zeros_f)
    return _tc3(hp_pre, lp_pre, p, t, H, U2, degp, LW, LB)


# speedup vs baseline: 1.0273x; 1.0273x over previous
import functools

import jax
import jax.numpy as jnp
from jax import lax
from jax.experimental import pallas as pl
from jax.experimental.pallas import tpu as pltpu
from jax.experimental.pallas import tpu_sc as plsc

N = 10000
D = 128
E = 320000
NC = 2
NS = 16
NW = NC * NS
EPW = E // NW
CH = 80
NCHK = EPW // CH
NPAD = 10240
RPS = NPAD // NS
ZR = 128
DEGN = 10240
RPSD = DEGN // NS


def _mesh():
    return plsc.VectorSubcoreMesh(
        core_axis_name="c", subcore_axis_name="s",
        num_cores=NC, num_subcores=NS)


def _make_deg():
    @functools.partial(
        pl.kernel,
        out_type=jax.ShapeDtypeStruct((NC, DEGN, D), jnp.float32),
        mesh=_mesh(),
        scratch_types=[
            pltpu.VMEM_SHARED((DEGN, D), jnp.float32),
            pltpu.VMEM((NCHK, CH), jnp.int32),
            pltpu.VMEM((CH, D), jnp.float32),
            pltpu.SemaphoreType.DMA,
        ],
    )
    def deg_k(dst_hbm, ones_hbm, zeros_hbm, out_hbm, acc, idx_v, ones_v,
              dsem):
        c = lax.axis_index("c")
        s = lax.axis_index("s")
        wid = c * NS + s
        pltpu.sync_copy(dst_hbm.at[wid], idx_v)
        pltpu.sync_copy(ones_hbm, ones_v)
        pltpu.sync_copy(zeros_hbm, acc.at[pl.ds(s * RPSD, RPSD)])
        plsc.subcore_barrier()

        def body(j, carry):
            pltpu.async_copy(ones_v, acc.at[idx_v.at[j]], dsem, add=True)
            return carry

        lax.fori_loop(0, NCHK, body, 0)

        def drain(j, carry):
            pltpu.make_async_copy(ones_v, acc.at[pl.ds(0, CH)], dsem).wait()
            return carry

        lax.fori_loop(0, NCHK, drain, 0)
        plsc.subcore_barrier()
        pltpu.sync_copy(acc.at[pl.ds(s * RPSD, RPSD)],
                        out_hbm.at[c, pl.ds(s * RPSD, RPSD)])

    return deg_k


NBUF = 3
RING = 6


def _make_spmm(F):
    @functools.partial(
        pl.kernel,
        out_type=jax.ShapeDtypeStruct((NC, F, NPAD, D), jnp.float32),
        mesh=_mesh(),
        scratch_types=[
            pltpu.VMEM_SHARED((NPAD, D), jnp.float32),
            pltpu.VMEM((RING, CH), jnp.int32),
            pltpu.VMEM((NCHK, CH), jnp.int32),
            pltpu.VMEM((NBUF, CH, D), jnp.float32),
            pltpu.SemaphoreType.DMA((NBUF,)),
            pltpu.SemaphoreType.DMA((NBUF,)),
            pltpu.SemaphoreType.DMA((RING,)),
        ],
    )
    def spmm_k(*args):
        hs = args[:F]
        src_hbm, dst_hbm, zeros_hbm, out_hbm = args[F:F + 4]
        acc, sring, dst_v, gbuf, gsem, ssem, isem = args[F + 4:]
        c = lax.axis_index("c")
        s = lax.axis_index("s")
        wid = c * NS + s
        pltpu.sync_copy(dst_hbm.at[wid], dst_v)

        def stage(i, slot):
            pltpu.async_copy(src_hbm.at[wid, pl.ds(i, 1)],
                             sring.at[pl.ds(slot, 1)], isem.at[slot])

        def wait_idx(slot):
            pltpu.make_async_copy(src_hbm.at[wid, pl.ds(0, 1)],
                                  sring.at[pl.ds(slot, 1)],
                                  isem.at[slot]).wait()

        def gather(h, slot, b):
            pltpu.async_copy(h.at[sring.at[slot]], gbuf.at[b], gsem.at[b])

        def wait_gather(h, b):
            pltpu.make_async_copy(h.at[pl.ds(0, CH)], gbuf.at[b],
                                  gsem.at[b]).wait()

        def scatter(i, b):
            pltpu.async_copy(gbuf.at[b], acc.at[dst_v.at[i]], ssem.at[b],
                             add=True)

        def wait_scatter(b):
            pltpu.make_async_copy(gbuf.at[b], acc.at[pl.ds(0, CH)],
                                  ssem.at[b]).wait()

        def step(h, i, k, i_static=True):
            b = k % NBUF
            b2 = (k + 2) % NBUF
            r = k % RING
            r2 = (k + 2) % RING
            if not (i_static and i < 1):
                wait_scatter(b2)
            if not (i_static and i + 2 >= NCHK):
                wait_idx(r2)
                gather(h, r2, b2)
            wait_gather(h, b)
            if not (i_static and i + 6 >= NCHK):
                stage(i + 6, r)
            scatter(i, b)

        for f in range(F):
            h = hs[f]
            pltpu.sync_copy(zeros_hbm, acc.at[pl.ds(s * RPS, RPS)])
            plsc.subcore_barrier()
            for r in range(RING):
                stage(r, r)
            wait_idx(0)
            gather(h, 0, 0)
            wait_idx(1)
            gather(h, 1, 1)
            for i in range(RING):
                step(h, i, i)
            def block(i6, carry):
                for k in range(RING):
                    step(h, i6 * RING + k, k, i_static=False)
                return carry
            n_main = (NCHK - 12) // RING
            lax.fori_loop(1, 1 + n_main, block, 0)
            for i in range((1 + n_main) * RING, NCHK):
                step(h, i, i % RING)
            wait_scatter((NCHK - 1) % NBUF)
            plsc.subcore_barrier()
            pltpu.sync_copy(acc.at[pl.ds(s * RPS, RPS)],
                            out_hbm.at[c, f, pl.ds(s * RPS, RPS)])
            plsc.subcore_barrier()

    return spmm_k


BN = 400


def _dinv_block(deg_ref):
    deg = deg_ref[0, :, 0] + deg_ref[1, :, 0]
    return lax.rsqrt(jnp.maximum(deg, 1.0))[:, None]


def _tc0_body(x_ref, w_ref, b_ref, h_ref):
    h_ref[...] = jnp.dot(x_ref[...], w_ref[...],
                         preferred_element_type=jnp.float32) + b_ref[...]


def _tc0(x, Wcat, bcat):
    nb = N // BN
    return pl.pallas_call(
        _tc0_body,
        grid=(nb,),
        in_specs=[
            pl.BlockSpec((BN, D), lambda i: (i, 0)),
            pl.BlockSpec((D, 5 * D), lambda i: (0, 0)),
            pl.BlockSpec((1, 5 * D), lambda i: (0, 0)),
        ],
        out_specs=pl.BlockSpec((BN, 5 * D), lambda i: (i, 0)),
        out_shape=jax.ShapeDtypeStruct((N, 5 * D), jnp.float32),
    )(x, Wcat, bcat)


def _tc1_body(h_ref, deg_ref, g0, g1, g2, g3):
    dinv = _dinv_block(deg_ref)
    for f, ref in enumerate((g0, g1, g2, g3)):
        ref[...] = h_ref[:, f * D:(f + 1) * D] * dinv


def _tc1(H, degp):
    nb = N // BN
    return pl.pallas_call(
        _tc1_body,
        grid=(nb,),
        in_specs=[
            pl.BlockSpec((BN, 5 * D), lambda i: (i, 0)),
            pl.BlockSpec((NC, BN, D), lambda i: (0, i, 0)),
        ],
        out_specs=[pl.BlockSpec((BN, D), lambda i: (i, 0))] * 4,
        out_shape=[jax.ShapeDtypeStruct((N, D), jnp.float32)] * 4,
    )(H, degp)


def _tc2_body(h_ref, u_ref, deg_ref, hp_pre, lp_pre, p_ref, t_ref, g2a, g2b):
    dinv = _dinv_block(deg_ref)
    u = u_ref[0] + u_ref[1]
    hp_pre[...] = h_ref[:, 0:D] - dinv * u[0]
    lp_pre[...] = dinv * u[1]
    pv = h_ref[:, 2 * D:3 * D] - dinv * u[2]
    tv = dinv * u[3]
    p_ref[...] = pv
    t_ref[...] = tv
    g2a[...] = dinv * pv
    g2b[...] = dinv * tv


def _tc2(H, U1, degp):
    nb = N // BN
    return pl.pallas_call(
        _tc2_body,
        grid=(nb,),
        in_specs=[
            pl.BlockSpec((BN, 5 * D), lambda i: (i, 0)),
            pl.BlockSpec((NC, 4, BN, D), lambda i: (0, 0, i, 0)),
            pl.BlockSpec((NC, BN, D), lambda i: (0, i, 0)),
        ],
        out_specs=[pl.BlockSpec((BN, D), lambda i: (i, 0))] * 6,
        out_shape=[jax.ShapeDtypeStruct((N, D), jnp.float32)] * 6,
    )(H, U1, degp)


def _tc3_body(hp_ref, lp_ref, p_ref, t_ref, h_ref, u_ref, deg_ref,
              lw_ref, lb_ref, out_ref):
    dinv = _dinv_block(deg_ref)
    u = u_ref[0] + u_ref[1]
    h_hp = jnp.maximum(hp_ref[...], 0.0)
    h_lp = jnp.maximum(lp_ref[...], 0.0)
    h_hp2 = jnp.maximum(p_ref[...] - dinv * u[0], 0.0)
    h_lp2 = jnp.maximum(dinv * u[1], 0.0)
    h_i = jnp.maximum(h_ref[:, 4 * D:5 * D], 0.0)
    acc = jnp.zeros((BN, D), jnp.float32)
    for k, hk in enumerate((h_hp, h_lp, h_hp2, h_lp2, h_i)):
        score = jnp.dot(hk, lw_ref[:, k:k + 1],
                        preferred_element_type=jnp.float32) + lb_ref[0, k]
        acc = acc + jax.nn.sigmoid(score) * hk
    out_ref[...] = acc


def _tc3(hp_pre, lp_pre, p, t, H, U2, degp, LW, LB):
    nb = N // BN
    return pl.pallas_call(
        _tc3_body,
        grid=(nb,),
        in_specs=[pl.BlockSpec((BN, D), lambda i: (i, 0))] * 4
        + [
            pl.BlockSpec((BN, 5 * D), lambda i: (i, 0)),
            pl.BlockSpec((NC, 2, BN, D), lambda i: (0, 0, i, 0)),
            pl.BlockSpec((NC, BN, D), lambda i: (0, i, 0)),
            pl.BlockSpec((D, 8), lambda i: (0, 0)),
            pl.BlockSpec((1, 8), lambda i: (0, 0)),
        ],
        out_specs=pl.BlockSpec((BN, D), lambda i: (i, 0)),
        out_shape=jax.ShapeDtypeStruct((N, D), jnp.float32),
    )(hp_pre, lp_pre, p, t, H, U2, degp, LW, LB)


def kernel(x, edge_index,
           W_hp, b_hp, lw_hp, lb_hp,
           W_lp, b_lp, lw_lp, lb_lp,
           W_i, b_i, lw_i, lb_i,
           W_hp2, b_hp2, lw_hp2, lb_hp2,
           W_lp2, b_lp2, lw_lp2, lb_lp2):
    src = edge_index[0].reshape(NW, NCHK, CH)
    dst = edge_index[1].reshape(NW, NCHK, CH)
    Wcat = jnp.concatenate([W_hp, W_lp, W_hp2, W_lp2, W_i], axis=1)
    bcat = jnp.concatenate([b_hp, b_lp, b_hp2, b_lp2, b_i]).reshape(1, 5 * D)
    LW = jnp.concatenate(
        [lw_hp, lw_lp, lw_hp2, lw_lp2, lw_i, jnp.zeros((D, 3), jnp.float32)],
        axis=1)
    LB = jnp.concatenate(
        [lb_hp, lb_lp, lb_hp2, lb_lp2, lb_i,
         jnp.zeros((3,), jnp.float32)]).reshape(1, 8)
    ones_c = jnp.ones((CH, D), jnp.float32)
    zeros_d = jnp.zeros((RPSD, D), jnp.float32)
    zeros_f = jnp.zeros((RPS, D), jnp.float32)

    degp = _make_deg()(dst, ones_c, zeros_d)
    H = _tc0(x, Wcat, bcat)
    g0, g1, g2, g3 = _tc1(H, degp)
    U1 = _make_spmm(4)(g0, g1, g2, g3, src, dst, zeros_f)
    hp_pre, lp_pre, p, t, g2a, g2b = _tc2(H, U1, degp)
    U2 = _make_spmm(2)(g2a, g2b, src, dst, zeros_f)
    return _tc3(hp_pre, lp_pre, p, t, H, U2, degp, LW, LB)
